# baseline (speedup 1.0000x reference)
import functools

import jax
import jax.numpy as jnp
import numpy as np
from jax import lax
from jax.experimental import pallas as pl
from jax.experimental.pallas import tpu as pltpu
from jax.experimental.pallas import tpu_sc as plsc

N = 10000
E = 320000
CZ = 128
CE = 16
H = 8
C = 16
D = 2 * CZ + CE

NBLK = 10
NB = N // NBLK
EBLK = 160
EB = E // EBLK

SROW = 288
DROW = 272
EROW = 400
PROW = 144
NPAD = 10112

NW = 32
BCH = 16
EPAD = 323584
EWP = EPAD // NW
NCH = EWP // BCH


def _a1_body(x_ref, p_ref, wq_ref, bq_ref, wkx_ref, wvx_ref, wkp_ref,
             wvp_ref, gk_ref, s_ref, src_ref, dst_ref):
    xb = x_ref[...]
    pb = p_ref[...]
    S = s_ref[...]
    q = jnp.dot(xb, wq_ref[...], preferred_element_type=jnp.float32) + bq_ref[...]
    Kx = jnp.dot(xb, wkx_ref[...], preferred_element_type=jnp.float32)
    Vx = jnp.dot(xb, wvx_ref[...], preferred_element_type=jnp.float32)
    Kp = jnp.dot(pb, wkp_ref[...], preferred_element_type=jnp.float32)
    Vp = jnp.dot(pb, wvp_ref[...], preferred_element_type=jnp.float32)
    dqKx = jnp.dot(q * Kx, S, preferred_element_type=jnp.float32)
    dqg = jnp.dot(q * gk_ref[...], S, preferred_element_type=jnp.float32)
    sx = jnp.sum(xb, axis=1, keepdims=True)
    ssx = jnp.sum(xb * xb, axis=1, keepdims=True)
    sp = jnp.sum(pb, axis=1, keepdims=True)
    ssp = jnp.sum(pb * pb, axis=1, keepdims=True)
    z8 = jnp.zeros((xb.shape[0], 8), jnp.float32)
    z6 = jnp.zeros((xb.shape[0], 6), jnp.float32)
    z14 = jnp.zeros((xb.shape[0], 14), jnp.float32)
    src_ref[:, 0:128] = q
    src_ref[:, 128:256] = Vx
    src_ref[:, 256:288] = jnp.concatenate([dqKx, z8, dqg, sx, ssx, z6], axis=1)
    dst_ref[:, 0:128] = Kp
    dst_ref[:, 128:256] = Vp
    dst_ref[:, 256:272] = jnp.concatenate([sp, ssp, z14], axis=1)


def _stage_a1(x, p, wq4, bq4, wkx, wvx, wkp, wvp, gk, S, interpret=False):
    full = lambda shp: pl.BlockSpec(shp, lambda i: (0,) * len(shp))
    return pl.pallas_call(
        _a1_body,
        grid=(NBLK,),
        in_specs=[
            pl.BlockSpec((NB, CZ), lambda i: (i, 0)),
            pl.BlockSpec((NB, CZ), lambda i: (i, 0)),
            full((CZ, H * C)), full((1, H * C)),
            full((CZ, H * C)), full((CZ, H * C)),
            full((CZ, H * C)), full((CZ, H * C)),
            full((1, H * C)), full((H * C, H)),
        ],
        out_specs=[
            pl.BlockSpec((NB, SROW), lambda i: (i, 0)),
            pl.BlockSpec((NB, DROW), lambda i: (i, 0)),
        ],
        out_shape=[
            jax.ShapeDtypeStruct((NPAD, SROW), jnp.float32),
            jax.ShapeDtypeStruct((NPAD, DROW), jnp.float32),
        ],
        interpret=interpret,
    )(x, p, wq4, bq4, wkx, wvx, wkp, wvp, gk, S)


def _a2_body(ea_ref, wka_ref, wva_ref, we_ref, ek_ref, ev_ref, ew_ref, ss_ref):
    eb = ea_ref[...]
    ek_ref[...] = jnp.dot(eb, wka_ref[...], preferred_element_type=jnp.float32)
    ev_ref[...] = jnp.dot(eb, wva_ref[...], preferred_element_type=jnp.float32)
    ew_ref[...] = jnp.dot(eb, we_ref[...], preferred_element_type=jnp.float32)
    sa = jnp.sum(eb, axis=1, keepdims=True)
    ssa = jnp.sum(eb * eb, axis=1, keepdims=True)
    z14 = jnp.zeros((eb.shape[0], 14), jnp.float32)
    ss_ref[...] = jnp.concatenate([sa, ssa, z14], axis=1)


def _stage_a2(ea, wka, wva, we, interpret=False):
    full = lambda shp: pl.BlockSpec(shp, lambda i: (0,) * len(shp))
    return pl.pallas_call(
        _a2_body,
        grid=(EBLK,),
        in_specs=[
            pl.BlockSpec((EB, CE), lambda i: (i, 0)),
            full((CE, H * C)), full((CE, H * C)), full((CE, H * C)),
        ],
        out_specs=[
            pl.BlockSpec((EB, H * C), lambda i: (i, 0)),
            pl.BlockSpec((EB, H * C), lambda i: (i, 0)),
            pl.BlockSpec((EB, H * C), lambda i: (i, 0)),
            pl.BlockSpec((EB, CE), lambda i: (i, 0)),
        ],
        out_shape=[
            jax.ShapeDtypeStruct((EPAD, H * C), jnp.float32),
            jax.ShapeDtypeStruct((EPAD, H * C), jnp.float32),
            jax.ShapeDtypeStruct((EPAD, H * C), jnp.float32),
            jax.ShapeDtypeStruct((EPAD, CE), jnp.float32),
        ],
        interpret=interpret,
    )(ea, wka, wva, we)


def _rsqrt16(v):
    i = lax.bitcast_convert_type(v, jnp.int32)
    i = jnp.int32(0x5F3759DF) - (i >> 1)
    y = lax.bitcast_convert_type(i, jnp.float32)
    for _ in range(3):
        y = y * (1.5 - 0.5 * v * y * y)
    return y


def _b_body(idx_hbm, stab_hbm, dtab_hbm, ek_hbm, ev_hbm, ew_hbm, ss_hbm,
            gv_hbm, cv_hbm,
            acc_hbm,
            idx0, idx1, srows, drows, ekb, evb, ewb, ssb, pay, gvv, cvv,
            acc_sh,
            isem0, isem1, ssem0, ssem1, dsem0, dsem1, lsem0, lsem1):
    c = lax.axis_index("c")
    s = lax.axis_index("s")
    wid = s * 2 + c
    lanes = lax.iota(jnp.int32, 16)

    pltpu.sync_copy(gv_hbm, gvv)
    pltpu.sync_copy(cv_hbm, cvv)

    def _zero(i, _):
        pay_z = jnp.zeros((16,), jnp.float32)
        for j in range(PROW // 16):
            pay[i, pl.ds(j * 16, 16)] = pay_z
        return 0
    lax.fori_loop(0, 2 * BCH, _zero, 0)
    myrows = NPAD // 16
    nz = myrows // (2 * BCH)
    for r in range(nz):
        pltpu.sync_copy(pay, acc_sh.at[pl.ds(s * myrows + r * 2 * BCH, 2 * BCH)])
    rem = myrows - nz * 2 * BCH
    pltpu.sync_copy(pay.at[pl.ds(0, rem)],
                    acc_sh.at[pl.ds(s * myrows + nz * 2 * BCH, rem)])

    gvs = tuple(gvv[h] for h in range(H))
    cvs = tuple(cvv[h] for h in range(H))

    idxs = (idx0, idx1)
    isems = (isem0, isem1)
    ssems = (ssem0, ssem1)
    dsems = (dsem0, dsem1)
    lsems = (lsem0, lsem1)
    row0 = wid * NCH
    ebase = wid * EWP

    def _issue_gathers(b, ch):
        off = b * BCH
        eoff = ebase + ch * BCH
        pltpu.async_copy(stab_hbm.at[idxs[b].at[0]],
                         srows.at[pl.ds(off, BCH)], ssems[b])
        pltpu.async_copy(dtab_hbm.at[idxs[b].at[1]],
                         drows.at[pl.ds(off, BCH)], dsems[b])
        pltpu.async_copy(ek_hbm.at[pl.ds(eoff, BCH)],
                         ekb.at[pl.ds(off, BCH)], lsems[b])
        pltpu.async_copy(ev_hbm.at[pl.ds(eoff, BCH)],
                         evb.at[pl.ds(off, BCH)], lsems[b])
        pltpu.async_copy(ew_hbm.at[pl.ds(eoff, BCH)],
                         ewb.at[pl.ds(off, BCH)], lsems[b])
        pltpu.async_copy(ss_hbm.at[pl.ds(eoff, BCH)],
                         ssb.at[pl.ds(off, BCH)], lsems[b])

    def _wait_gathers(b):
        off = b * BCH
        pltpu.make_async_copy(stab_hbm.at[pl.ds(0, BCH)],
                              srows.at[pl.ds(off, BCH)], ssems[b]).wait()
        pltpu.make_async_copy(dtab_hbm.at[pl.ds(0, BCH)],
                              drows.at[pl.ds(off, BCH)], dsems[b]).wait()
        pltpu.make_async_copy(ek_hbm.at[pl.ds(0, BCH)],
                              ekb.at[pl.ds(off, BCH)], lsems[b]).wait()
        pltpu.make_async_copy(ev_hbm.at[pl.ds(0, BCH)],
                              evb.at[pl.ds(off, BCH)], lsems[b]).wait()
        pltpu.make_async_copy(ew_hbm.at[pl.ds(0, BCH)],
                              ewb.at[pl.ds(off, BCH)], lsems[b]).wait()
        pltpu.make_async_copy(ss_hbm.at[pl.ds(0, BCH)],
                              ssb.at[pl.ds(off, BCH)], lsems[b]).wait()

    plsc.subcore_barrier()

    pltpu.sync_copy(idx_hbm.at[row0], idx0)
    pltpu.async_copy(idx_hbm.at[row0 + 1], idx1, isem1)
    _issue_gathers(0, 0)

    def _compute(boff):
        rowv = lanes + boff

        def ld(ref, col):
            return plsc.load_gather(
                ref, [rowv, jnp.full((16,), col, jnp.int32)])

        sx = ld(srows, 280)
        ssx = ld(srows, 281)
        sp = ld(drows, 256)
        ssp = ld(drows, 257)
        sa = ld(ssb, 0)
        ssa = ld(ssb, 1)
        mu = (sa + sx + sp) * (1.0 / D)
        msq = (ssa + ssx + ssp) * (1.0 / D)
        rstd = _rsqrt16(msq - mu * mu + 1e-5)
        rm = rstd * mu
        for h in range(H):
            acc0 = ld(srows, 256 + h) - mu * ld(srows, 272 + h)
            for ci in range(16):
                col = h * 16 + ci
                acc0 = acc0 + ld(srows, col) * (ld(ekb, col) + ld(drows, col))
            exh = jnp.exp(acc0 * rstd)
            plsc.store_scatter(
                pay, [rowv, jnp.full((16,), 128 + h, jnp.int32)], exh)
            for ci in range(16):
                col = h * 16 + ci
                U = ld(evb, col) + ld(drows, 128 + col) + ld(srows, 128 + col)
                vv = rstd * U - rm * gvs[h][ci] + cvs[h][ci]
                plsc.store_scatter(
                    pay, [rowv, jnp.full((16,), col, jnp.int32)],
                    exh * ld(ewb, col) * vv)

    def _pair(it, _):
        for b in (0, 1):
            ch = 2 * it + b
            nb = 1 - b

            @pl.when(ch + 1 < NCH)
            def _prep():
                pltpu.make_async_copy(idx_hbm.at[row0], idxs[nb],
                                      isems[nb]).wait()
                _issue_gathers(nb, ch + 1)

            _wait_gathers(b)
            _compute(b * BCH)
            pltpu.sync_copy(pay.at[pl.ds(b * BCH, BCH)],
                            acc_sh.at[idxs[b].at[0]], add=True)

            @pl.when(ch + 2 < NCH)
            def _pref_idx():
                pltpu.async_copy(idx_hbm.at[row0 + ch + 2], idxs[b], isems[b])
        return 0

    lax.fori_loop(0, NCH // 2, _pair, 0)
    plsc.subcore_barrier()
    pltpu.sync_copy(acc_sh.at[pl.ds(s * myrows, myrows)],
                    acc_hbm.at[c, pl.ds(s * myrows, myrows)])


def _stage_b(idx3, stab, dtab, ek, ev, ew, ss, gv, cv):
    mesh = plsc.VectorSubcoreMesh(core_axis_name="c", subcore_axis_name="s")
    kfn = pl.kernel(
        _b_body,
        mesh=mesh,
        compiler_params=pltpu.CompilerParams(
            needs_layout_passes=False, use_tc_tiling_on_sc=False),
        out_type=jax.ShapeDtypeStruct((2, NPAD, PROW), jnp.float32),
        scratch_types=[
            pltpu.VMEM((2, BCH), jnp.int32),
            pltpu.VMEM((2, BCH), jnp.int32),
            pltpu.VMEM((2 * BCH, SROW), jnp.float32),
            pltpu.VMEM((2 * BCH, DROW), jnp.float32),
            pltpu.VMEM((2 * BCH, H * C), jnp.float32),
            pltpu.VMEM((2 * BCH, H * C), jnp.float32),
            pltpu.VMEM((2 * BCH, H * C), jnp.float32),
            pltpu.VMEM((2 * BCH, CE), jnp.float32),
            pltpu.VMEM((2 * BCH, PROW), jnp.float32),
            pltpu.VMEM((H, 16), jnp.float32),
            pltpu.VMEM((H, 16), jnp.float32),
            pltpu.VMEM_SHARED((NPAD, PROW), jnp.float32),
            pltpu.SemaphoreType.DMA,
            pltpu.SemaphoreType.DMA,
            pltpu.SemaphoreType.DMA,
            pltpu.SemaphoreType.DMA,
            pltpu.SemaphoreType.DMA,
            pltpu.SemaphoreType.DMA,
            pltpu.SemaphoreType.DMA,
            pltpu.SemaphoreType.DMA,
        ],
    )
    return kfn(idx3, stab, dtab, ek, ev, ew, ss, gv, cv)


def _c_body(acc_ref, r_ref, wo1_ref, bo1_ref, wo2_ref, bo2_ref, out_ref):
    asum = acc_ref[0] + acc_ref[1]
    num = asum[:, 0:128]
    den = asum[:, 128:136]
    denb = jnp.dot(den, r_ref[...], preferred_element_type=jnp.float32)
    out0 = jnp.where(denb != 0.0, num / denb, 0.0)
    h1 = jnp.dot(out0, wo1_ref[...], preferred_element_type=jnp.float32) + bo1_ref[...]
    h1 = h1 * (1.0 / (1.0 + jnp.exp(-h1)))
    out_ref[...] = jnp.dot(h1, wo2_ref[...], preferred_element_type=jnp.float32) + bo2_ref[...]


def _stage_c(acc, Rm, wo1, bo1, wo2, bo2, interpret=False):
    bo1 = bo1.reshape(1, 2 * CZ)
    bo2 = bo2.reshape(1, CZ)
    full = lambda shp: pl.BlockSpec(shp, lambda i: (0,) * len(shp))
    return pl.pallas_call(
        _c_body,
        grid=(NBLK,),
        in_specs=[
            pl.BlockSpec((2, NB, PROW), lambda i: (0, i, 0)),
            full((H, H * C)),
            full((CZ, 2 * CZ)), full((1, 2 * CZ)),
            full((2 * CZ, CZ)), full((1, CZ)),
        ],
        out_specs=pl.BlockSpec((NB, CZ), lambda i: (i, 0)),
        out_shape=jax.ShapeDtypeStruct((N, CZ), jnp.float32),
        interpret=interpret,
    )(acc, Rm, wo1, bo1, wo2, bo2)


def kernel(x, p, edge_index, edge_attr, ln_gamma, ln_beta, Wq, bq, Wk, bk,
           Wv, bv, We, Wo1, bo1, Wo2, bo2):
    f32 = jnp.float32
    Wkg = ln_gamma[:, None] * Wk
    Wvg = ln_gamma[:, None] * Wv
    scale = f32(1.0 / np.sqrt(C))
    wq4 = Wq * scale
    bq4 = (bq * scale).reshape(1, H * C)
    wka, wkx, wkp = Wkg[:CE], Wkg[CE:CE + CZ], Wkg[CE + CZ:]
    wva, wvx, wvp = Wvg[:CE], Wvg[CE:CE + CZ], Wvg[CE + CZ:]
    gk = (ln_gamma @ Wk).reshape(1, H * C)
    gv = (ln_gamma @ Wv).reshape(H, C)
    cv = (ln_beta @ Wv + bv).reshape(H, C)
    ii = np.arange(H * C)
    S = jnp.asarray((ii[:, None] // C == np.arange(H)[None, :]).astype(np.float32))
    Rm = jnp.asarray((ii[None, :] // C == np.arange(H)[:, None]).astype(np.float32))

    src = jnp.concatenate([edge_index[0].astype(jnp.int32),
                           jnp.full((EPAD - E,), NPAD - 1, jnp.int32)])
    dst = jnp.concatenate([edge_index[1].astype(jnp.int32),
                           jnp.zeros((EPAD - E,), jnp.int32)])
    idx3 = jnp.stack([src.reshape(-1, BCH), dst.reshape(-1, BCH)], axis=1)

    stab, dtab = _stage_a1(x, p, wq4, bq4, wkx, wvx, wkp, wvp, gk, S)
    ek, ev, ew, ss = _stage_a2(edge_attr, wka, wva, We)
    acc = _stage_b(idx3, stab, dtab, ek, ev, ew, ss, gv, cv)
    return _stage_c(acc, Rm, Wo1, bo1, Wo2, bo2)


# speedup vs baseline: 2.5256x; 2.5256x over previous
import functools

import jax
import jax.numpy as jnp
import numpy as np
from jax import lax
from jax.experimental import pallas as pl
from jax.experimental.pallas import tpu as pltpu
from jax.experimental.pallas import tpu_sc as plsc

N = 10000
E = 320000
CZ = 128
CE = 16
H = 8
C = 16
D = 2 * CZ + CE

NBLK = 10
NB = N // NBLK
EBLK = 160
EB = E // EBLK

SROW = 288
DROW = 272
EROW = 400
PROW = 144
NPAD = 10112

NW = 32
BCH = 16
EPAD = 323584
EWP = EPAD // NW
NCH = EWP // BCH


def _a1_body(x_ref, p_ref, wq_ref, bq_ref, wkx_ref, wvx_ref, wkp_ref,
             wvp_ref, gk_ref, s_ref, src_ref, dst_ref):
    xb = x_ref[...]
    pb = p_ref[...]
    S = s_ref[...]
    q = jnp.dot(xb, wq_ref[...], preferred_element_type=jnp.float32) + bq_ref[...]
    Kx = jnp.dot(xb, wkx_ref[...], preferred_element_type=jnp.float32)
    Vx = jnp.dot(xb, wvx_ref[...], preferred_element_type=jnp.float32)
    Kp = jnp.dot(pb, wkp_ref[...], preferred_element_type=jnp.float32)
    Vp = jnp.dot(pb, wvp_ref[...], preferred_element_type=jnp.float32)
    dqKx = jnp.dot(q * Kx, S, preferred_element_type=jnp.float32)
    dqg = jnp.dot(q * gk_ref[...], S, preferred_element_type=jnp.float32)
    sx = jnp.sum(xb, axis=1, keepdims=True)
    ssx = jnp.sum(xb * xb, axis=1, keepdims=True)
    sp = jnp.sum(pb, axis=1, keepdims=True)
    ssp = jnp.sum(pb * pb, axis=1, keepdims=True)
    z8 = jnp.zeros((xb.shape[0], 8), jnp.float32)
    z6 = jnp.zeros((xb.shape[0], 6), jnp.float32)
    z14 = jnp.zeros((xb.shape[0], 14), jnp.float32)
    src_ref[:, 0:128] = q
    src_ref[:, 128:256] = Vx
    src_ref[:, 256:288] = jnp.concatenate([dqKx, z8, dqg, sx, ssx, z6], axis=1)
    dst_ref[:, 0:128] = Kp
    dst_ref[:, 128:256] = Vp
    dst_ref[:, 256:272] = jnp.concatenate([sp, ssp, z14], axis=1)


def _stage_a1(x, p, wq4, bq4, wkx, wvx, wkp, wvp, gk, S, interpret=False):
    full = lambda shp: pl.BlockSpec(shp, lambda i: (0,) * len(shp))
    return pl.pallas_call(
        _a1_body,
        grid=(NBLK,),
        in_specs=[
            pl.BlockSpec((NB, CZ), lambda i: (i, 0)),
            pl.BlockSpec((NB, CZ), lambda i: (i, 0)),
            full((CZ, H * C)), full((1, H * C)),
            full((CZ, H * C)), full((CZ, H * C)),
            full((CZ, H * C)), full((CZ, H * C)),
            full((1, H * C)), full((H * C, H)),
        ],
        out_specs=[
            pl.BlockSpec((NB, SROW), lambda i: (i, 0)),
            pl.BlockSpec((NB, DROW), lambda i: (i, 0)),
        ],
        out_shape=[
            jax.ShapeDtypeStruct((NPAD, SROW), jnp.float32),
            jax.ShapeDtypeStruct((NPAD, DROW), jnp.float32),
        ],
        interpret=interpret,
    )(x, p, wq4, bq4, wkx, wvx, wkp, wvp, gk, S)


def _a2_body(ea_ref, wka_ref, wva_ref, we_ref, ek_ref, ev_ref, ew_ref, ss_ref):
    eb = ea_ref[...]
    ek_ref[...] = jnp.dot(eb, wka_ref[...], preferred_element_type=jnp.float32)
    ev_ref[...] = jnp.dot(eb, wva_ref[...], preferred_element_type=jnp.float32)
    ew_ref[...] = jnp.dot(eb, we_ref[...], preferred_element_type=jnp.float32)
    sa = jnp.sum(eb, axis=1, keepdims=True)
    ssa = jnp.sum(eb * eb, axis=1, keepdims=True)
    z14 = jnp.zeros((eb.shape[0], 14), jnp.float32)
    ss_ref[...] = jnp.concatenate([sa, ssa, z14], axis=1)


def _stage_a2(ea, wka, wva, we, interpret=False):
    full = lambda shp: pl.BlockSpec(shp, lambda i: (0,) * len(shp))
    return pl.pallas_call(
        _a2_body,
        grid=(EBLK,),
        in_specs=[
            pl.BlockSpec((EB, CE), lambda i: (i, 0)),
            full((CE, H * C)), full((CE, H * C)), full((CE, H * C)),
        ],
        out_specs=[
            pl.BlockSpec((EB, H * C), lambda i: (i, 0)),
            pl.BlockSpec((EB, H * C), lambda i: (i, 0)),
            pl.BlockSpec((EB, H * C), lambda i: (i, 0)),
            pl.BlockSpec((EB, CE), lambda i: (i, 0)),
        ],
        out_shape=[
            jax.ShapeDtypeStruct((EPAD, H * C), jnp.float32),
            jax.ShapeDtypeStruct((EPAD, H * C), jnp.float32),
            jax.ShapeDtypeStruct((EPAD, H * C), jnp.float32),
            jax.ShapeDtypeStruct((EPAD, CE), jnp.float32),
        ],
        interpret=interpret,
    )(ea, wka, wva, we)


def _rsqrt16(v):
    i = lax.bitcast_convert_type(v, jnp.int32)
    i = jnp.int32(0x5F3759DF) - (i >> 1)
    y = lax.bitcast_convert_type(i, jnp.float32)
    for _ in range(3):
        y = y * (1.5 - 0.5 * v * y * y)
    return y


def _b_body(idx_hbm, stab_hbm, dtab_hbm, ek_hbm, ev_hbm, ew_hbm, ss_hbm,
            gv_hbm, cv_hbm,
            acc_hbm,
            idx0, idx1, srows, drows, ekb, evb, ewb, ssb, pay, gvv, cvv,
            acc_sh,
            isem0, isem1, ssem0, ssem1, dsem0, dsem1, lsem0, lsem1):
    c = lax.axis_index("c")
    s = lax.axis_index("s")
    wid = s * 2 + c
    lanes = lax.iota(jnp.int32, 16)

    pltpu.sync_copy(gv_hbm, gvv)
    pltpu.sync_copy(cv_hbm, cvv)

    def _zero(i, _):
        pay_z = jnp.zeros((16,), jnp.float32)
        for j in range(PROW // 16):
            pay[i, pl.ds(j * 16, 16)] = pay_z
        return 0
    lax.fori_loop(0, 2 * BCH, _zero, 0)
    myrows = NPAD // 16
    nz = myrows // (2 * BCH)
    for r in range(nz):
        pltpu.sync_copy(pay, acc_sh.at[pl.ds(s * myrows + r * 2 * BCH, 2 * BCH)])
    rem = myrows - nz * 2 * BCH
    pltpu.sync_copy(pay.at[pl.ds(0, rem)],
                    acc_sh.at[pl.ds(s * myrows + nz * 2 * BCH, rem)])

    gvs = tuple(gvv[h] for h in range(H))
    cvs = tuple(cvv[h] for h in range(H))

    idxs = (idx0, idx1)
    isems = (isem0, isem1)
    ssems = (ssem0, ssem1)
    dsems = (dsem0, dsem1)
    lsems = (lsem0, lsem1)
    row0 = wid * NCH
    ebase = wid * EWP

    def _issue_gathers(b, ch):
        off = b * BCH
        eoff = ebase + ch * BCH
        pltpu.async_copy(stab_hbm.at[idxs[b].at[0]],
                         srows.at[pl.ds(off, BCH)], ssems[b])
        pltpu.async_copy(dtab_hbm.at[idxs[b].at[1]],
                         drows.at[pl.ds(off, BCH)], dsems[b])
        pltpu.async_copy(ek_hbm.at[pl.ds(eoff, BCH)],
                         ekb.at[pl.ds(off, BCH)], lsems[b])
        pltpu.async_copy(ev_hbm.at[pl.ds(eoff, BCH)],
                         evb.at[pl.ds(off, BCH)], lsems[b])
        pltpu.async_copy(ew_hbm.at[pl.ds(eoff, BCH)],
                         ewb.at[pl.ds(off, BCH)], lsems[b])
        pltpu.async_copy(ss_hbm.at[pl.ds(eoff, BCH)],
                         ssb.at[pl.ds(off, BCH)], lsems[b])

    def _wait_gathers(b):
        off = b * BCH
        pltpu.make_async_copy(stab_hbm.at[pl.ds(0, BCH)],
                              srows.at[pl.ds(off, BCH)], ssems[b]).wait()
        pltpu.make_async_copy(dtab_hbm.at[pl.ds(0, BCH)],
                              drows.at[pl.ds(off, BCH)], dsems[b]).wait()
        pltpu.make_async_copy(ek_hbm.at[pl.ds(0, BCH)],
                              ekb.at[pl.ds(off, BCH)], lsems[b]).wait()
        pltpu.make_async_copy(ev_hbm.at[pl.ds(0, BCH)],
                              evb.at[pl.ds(off, BCH)], lsems[b]).wait()
        pltpu.make_async_copy(ew_hbm.at[pl.ds(0, BCH)],
                              ewb.at[pl.ds(off, BCH)], lsems[b]).wait()
        pltpu.make_async_copy(ss_hbm.at[pl.ds(0, BCH)],
                              ssb.at[pl.ds(off, BCH)], lsems[b]).wait()

    plsc.subcore_barrier()

    pltpu.sync_copy(idx_hbm.at[row0], idx0)
    pltpu.async_copy(idx_hbm.at[row0 + 1], idx1, isem1)
    _issue_gathers(0, 0)

    def _edge(boff, e):
        r = boff + e
        dq2 = srows[r, pl.ds(272, 16)]
        dvec = drows[r, pl.ds(256, 16)]
        evec = ssb[r, pl.ds(0, 16)]
        mu = jnp.full((16,), (evec[0] + dq2[8] + dvec[0]) * (1.0 / D))
        msq = jnp.full((16,), (evec[1] + dq2[9] + dvec[1]) * (1.0 / D))
        rstd = _rsqrt16(msq - mu * mu + 1e-5)
        sv = jnp.zeros((16,), jnp.float32)
        for h in range(H):
            qh = srows[r, pl.ds(h * 16, 16)]
            kh = ekb[r, pl.ds(h * 16, 16)] + drows[r, pl.ds(h * 16, 16)]
            sh = jnp.sum(qh * kh)
            sv = jnp.where(lanes == h, jnp.full((16,), sh), sv)
        dq1 = srows[r, pl.ds(256, 16)]
        score = (sv + dq1 - mu * dq2) * rstd
        ex = jnp.where(lanes < H, jnp.exp(score), 0.0)
        pay[r, pl.ds(128, 16)] = ex
        rm = rstd * mu
        for h in range(H):
            Uh = (evb[r, pl.ds(h * 16, 16)]
                  + drows[r, pl.ds(128 + h * 16, 16)]
                  + srows[r, pl.ds(128 + h * 16, 16)])
            vvh = rstd * Uh - rm * gvs[h] + cvs[h]
            exh = jnp.full((16,), ex[h])
            pay[r, pl.ds(h * 16, 16)] = exh * ewb[r, pl.ds(h * 16, 16)] * vvh

    def _pair(it, _):
        for b in (0, 1):
            ch = 2 * it + b
            nb = 1 - b

            @pl.when(ch + 1 < NCH)
            def _prep():
                pltpu.make_async_copy(idx_hbm.at[row0], idxs[nb],
                                      isems[nb]).wait()
                _issue_gathers(nb, ch + 1)

            _wait_gathers(b)
            boff = b * BCH

            def _edge_b(e, _, boff=boff):
                _edge(boff, e)
                return 0
            lax.fori_loop(0, BCH, _edge_b, 0, unroll=4)
            pltpu.sync_copy(pay.at[pl.ds(boff, BCH)],
                            acc_sh.at[idxs[b].at[0]], add=True)

            @pl.when(ch + 2 < NCH)
            def _pref_idx():
                pltpu.async_copy(idx_hbm.at[row0 + ch + 2], idxs[b], isems[b])
        return 0

    lax.fori_loop(0, NCH // 2, _pair, 0)
    plsc.subcore_barrier()
    pltpu.sync_copy(acc_sh.at[pl.ds(s * myrows, myrows)],
                    acc_hbm.at[c, pl.ds(s * myrows, myrows)])


def _stage_b(idx3, stab, dtab, ek, ev, ew, ss, gv, cv):
    mesh = plsc.VectorSubcoreMesh(core_axis_name="c", subcore_axis_name="s")
    kfn = pl.kernel(
        _b_body,
        mesh=mesh,
        compiler_params=pltpu.CompilerParams(
            needs_layout_passes=False, use_tc_tiling_on_sc=False),
        out_type=jax.ShapeDtypeStruct((2, NPAD, PROW), jnp.float32),
        scratch_types=[
            pltpu.VMEM((2, BCH), jnp.int32),
            pltpu.VMEM((2, BCH), jnp.int32),
            pltpu.VMEM((2 * BCH, SROW), jnp.float32),
            pltpu.VMEM((2 * BCH, DROW), jnp.float32),
            pltpu.VMEM((2 * BCH, H * C), jnp.float32),
            pltpu.VMEM((2 * BCH, H * C), jnp.float32),
            pltpu.VMEM((2 * BCH, H * C), jnp.float32),
            pltpu.VMEM((2 * BCH, CE), jnp.float32),
            pltpu.VMEM((2 * BCH, PROW), jnp.float32),
            pltpu.VMEM((H, 16), jnp.float32),
            pltpu.VMEM((H, 16), jnp.float32),
            pltpu.VMEM_SHARED((NPAD, PROW), jnp.float32),
            pltpu.SemaphoreType.DMA,
            pltpu.SemaphoreType.DMA,
            pltpu.SemaphoreType.DMA,
            pltpu.SemaphoreType.DMA,
            pltpu.SemaphoreType.DMA,
            pltpu.SemaphoreType.DMA,
            pltpu.SemaphoreType.DMA,
            pltpu.SemaphoreType.DMA,
        ],
    )
    return kfn(idx3, stab, dtab, ek, ev, ew, ss, gv, cv)


def _c_body(acc_ref, r_ref, wo1_ref, bo1_ref, wo2_ref, bo2_ref, out_ref):
    asum = acc_ref[0] + acc_ref[1]
    num = asum[:, 0:128]
    den = asum[:, 128:136]
    denb = jnp.dot(den, r_ref[...], preferred_element_type=jnp.float32)
    out0 = jnp.where(denb != 0.0, num / denb, 0.0)
    h1 = jnp.dot(out0, wo1_ref[...], preferred_element_type=jnp.float32) + bo1_ref[...]
    h1 = h1 * (1.0 / (1.0 + jnp.exp(-h1)))
    out_ref[...] = jnp.dot(h1, wo2_ref[...], preferred_element_type=jnp.float32) + bo2_ref[...]


def _stage_c(acc, Rm, wo1, bo1, wo2, bo2, interpret=False):
    bo1 = bo1.reshape(1, 2 * CZ)
    bo2 = bo2.reshape(1, CZ)
    full = lambda shp: pl.BlockSpec(shp, lambda i: (0,) * len(shp))
    return pl.pallas_call(
        _c_body,
        grid=(NBLK,),
        in_specs=[
            pl.BlockSpec((2, NB, PROW), lambda i: (0, i, 0)),
            full((H, H * C)),
            full((CZ, 2 * CZ)), full((1, 2 * CZ)),
            full((2 * CZ, CZ)), full((1, CZ)),
        ],
        out_specs=pl.BlockSpec((NB, CZ), lambda i: (i, 0)),
        out_shape=jax.ShapeDtypeStruct((N, CZ), jnp.float32),
        interpret=interpret,
    )(acc, Rm, wo1, bo1, wo2, bo2)


def kernel(x, p, edge_index, edge_attr, ln_gamma, ln_beta, Wq, bq, Wk, bk,
           Wv, bv, We, Wo1, bo1, Wo2, bo2):
    f32 = jnp.float32
    Wkg = ln_gamma[:, None] * Wk
    Wvg = ln_gamma[:, None] * Wv
    scale = f32(1.0 / np.sqrt(C))
    wq4 = Wq * scale
    bq4 = (bq * scale).reshape(1, H * C)
    wka, wkx, wkp = Wkg[:CE], Wkg[CE:CE + CZ], Wkg[CE + CZ:]
    wva, wvx, wvp = Wvg[:CE], Wvg[CE:CE + CZ], Wvg[CE + CZ:]
    gk = (ln_gamma @ Wk).reshape(1, H * C)
    gv = (ln_gamma @ Wv).reshape(H, C)
    cv = (ln_beta @ Wv + bv).reshape(H, C)
    ii = np.arange(H * C)
    S = jnp.asarray((ii[:, None] // C == np.arange(H)[None, :]).astype(np.float32))
    Rm = jnp.asarray((ii[None, :] // C == np.arange(H)[:, None]).astype(np.float32))

    src = jnp.concatenate([edge_index[0].astype(jnp.int32),
                           jnp.full((EPAD - E,), NPAD - 1, jnp.int32)])
    dst = jnp.concatenate([edge_index[1].astype(jnp.int32),
                           jnp.zeros((EPAD - E,), jnp.int32)])
    idx3 = jnp.stack([src.reshape(-1, BCH), dst.reshape(-1, BCH)], axis=1)

    stab, dtab = _stage_a1(x, p, wq4, bq4, wkx, wvx, wkp, wvp, gk, S)
    ek, ev, ew, ss = _stage_a2(edge_attr, wka, wva, We)
    acc = _stage_b(idx3, stab, dtab, ek, ev, ew, ss, gv, cv)
    return _stage_c(acc, Rm, Wo1, bo1, Wo2, bo2)


# speedup vs baseline: 2.5308x; 1.0020x over previous
import jax
import jax.numpy as jnp
import numpy as np
from jax import lax
from jax.experimental import pallas as pl
from jax.experimental.pallas import tpu as pltpu
from jax.experimental.pallas import tpu_sc as plsc

N = 10000
E = 320000
CZ = 128
CE = 16
H = 8
C = 16
D = 2 * CZ + CE

NBLK = 10
NB = N // NBLK
EBLK = 160
EB = E // EBLK

SROW = 288
DROW = 272
PROW = 144
NPAD = 10112

NW = 32
BCH = 16
EPAD = 323584
EWP = EPAD // NW
NCH = EWP // BCH


def _a1_body(x_ref, p_ref, wq_ref, bq_ref, wkx_ref, wvx_ref, wkp_ref,
             wvp_ref, gk_ref, s_ref, src_ref, dst_ref):
    xb = x_ref[...]
    pb = p_ref[...]
    S = s_ref[...]
    q = jnp.dot(xb, wq_ref[...], preferred_element_type=jnp.float32) + bq_ref[...]
    Kx = jnp.dot(xb, wkx_ref[...], preferred_element_type=jnp.float32)
    Vx = jnp.dot(xb, wvx_ref[...], preferred_element_type=jnp.float32)
    Kp = jnp.dot(pb, wkp_ref[...], preferred_element_type=jnp.float32)
    Vp = jnp.dot(pb, wvp_ref[...], preferred_element_type=jnp.float32)
    dqKx = jnp.dot(q * Kx, S, preferred_element_type=jnp.float32)
    dqg = jnp.dot(q * gk_ref[...], S, preferred_element_type=jnp.float32)
    sx = jnp.sum(xb, axis=1, keepdims=True)
    ssx = jnp.sum(xb * xb, axis=1, keepdims=True)
    sp = jnp.sum(pb, axis=1, keepdims=True)
    ssp = jnp.sum(pb * pb, axis=1, keepdims=True)
    z8 = jnp.zeros((xb.shape[0], 8), jnp.float32)
    z6 = jnp.zeros((xb.shape[0], 6), jnp.float32)
    z14 = jnp.zeros((xb.shape[0], 14), jnp.float32)
    src_ref[:, 0:128] = q
    src_ref[:, 128:256] = Vx
    src_ref[:, 256:288] = jnp.concatenate([dqKx, z8, dqg, sx, ssx, z6], axis=1)
    dst_ref[:, 0:128] = Kp
    dst_ref[:, 128:256] = Vp
    dst_ref[:, 256:272] = jnp.concatenate([sp, ssp, z14], axis=1)


def _stage_a1(x, p, wq4, bq4, wkx, wvx, wkp, wvp, gk, S, interpret=False):
    full = lambda shp: pl.BlockSpec(shp, lambda i: (0,) * len(shp))
    return pl.pallas_call(
        _a1_body,
        grid=(NBLK,),
        in_specs=[
            pl.BlockSpec((NB, CZ), lambda i: (i, 0)),
            pl.BlockSpec((NB, CZ), lambda i: (i, 0)),
            full((CZ, H * C)), full((1, H * C)),
            full((CZ, H * C)), full((CZ, H * C)),
            full((CZ, H * C)), full((CZ, H * C)),
            full((1, H * C)), full((H * C, H)),
        ],
        out_specs=[
            pl.BlockSpec((NB, SROW), lambda i: (i, 0)),
            pl.BlockSpec((NB, DROW), lambda i: (i, 0)),
        ],
        out_shape=[
            jax.ShapeDtypeStruct((NPAD, SROW), jnp.float32),
            jax.ShapeDtypeStruct((NPAD, DROW), jnp.float32),
        ],
        interpret=interpret,
    )(x, p, wq4, bq4, wkx, wvx, wkp, wvp, gk, S)


def _a2_body(ea_ref, wka_ref, wva_ref, we_ref, ek_ref, ev_ref, ew_ref, ss_ref):
    eb = ea_ref[...]
    ek_ref[...] = jnp.dot(eb, wka_ref[...], preferred_element_type=jnp.float32)
    ev_ref[...] = jnp.dot(eb, wva_ref[...], preferred_element_type=jnp.float32)
    ew_ref[...] = jnp.dot(eb, we_ref[...], preferred_element_type=jnp.float32)
    sa = jnp.sum(eb, axis=1, keepdims=True)
    ssa = jnp.sum(eb * eb, axis=1, keepdims=True)
    z14 = jnp.zeros((eb.shape[0], 14), jnp.float32)
    ss_ref[...] = jnp.concatenate([sa, ssa, z14], axis=1)


def _stage_a2(ea, wka, wva, we, interpret=False):
    full = lambda shp: pl.BlockSpec(shp, lambda i: (0,) * len(shp))
    return pl.pallas_call(
        _a2_body,
        grid=(EBLK,),
        in_specs=[
            pl.BlockSpec((EB, CE), lambda i: (i, 0)),
            full((CE, H * C)), full((CE, H * C)), full((CE, H * C)),
        ],
        out_specs=[
            pl.BlockSpec((EB, H * C), lambda i: (i, 0)),
            pl.BlockSpec((EB, H * C), lambda i: (i, 0)),
            pl.BlockSpec((EB, H * C), lambda i: (i, 0)),
            pl.BlockSpec((EB, CE), lambda i: (i, 0)),
        ],
        out_shape=[
            jax.ShapeDtypeStruct((EPAD, H * C), jnp.float32),
            jax.ShapeDtypeStruct((EPAD, H * C), jnp.float32),
            jax.ShapeDtypeStruct((EPAD, H * C), jnp.float32),
            jax.ShapeDtypeStruct((EPAD, CE), jnp.float32),
        ],
        interpret=interpret,
    )(ea, wka, wva, we)


def _rsqrt16(v):
    i = lax.bitcast_convert_type(v, jnp.int32)
    i = jnp.int32(0x5F3759DF) - (i >> 1)
    y = lax.bitcast_convert_type(i, jnp.float32)
    for _ in range(3):
        y = y * (1.5 - 0.5 * v * y * y)
    return y


def _b_body(idx_hbm, stab_hbm, dtab_hbm, ek_hbm, ev_hbm, ew_hbm, ss_hbm,
            gv_hbm, cv_hbm,
            acc_hbm,
            idx0, idx1, srows, drows, ekb, evb, ewb, ssb, pay, gvv, cvv,
            acc_sh,
            isem0, isem1, ssem0, ssem1, dsem0, dsem1, lsem0, lsem1):
    c = lax.axis_index("c")
    s = lax.axis_index("s")
    wid = s * 2 + c
    lanes = lax.iota(jnp.int32, 16)

    pltpu.sync_copy(gv_hbm, gvv)
    pltpu.sync_copy(cv_hbm, cvv)

    def _zero(i, _):
        pay_z = jnp.zeros((16,), jnp.float32)
        for j in range(PROW // 16):
            pay[i, pl.ds(j * 16, 16)] = pay_z
        return 0
    lax.fori_loop(0, 2 * BCH, _zero, 0)
    myrows = NPAD // 16
    nz = myrows // (2 * BCH)
    for r in range(nz):
        pltpu.sync_copy(pay, acc_sh.at[pl.ds(s * myrows + r * 2 * BCH, 2 * BCH)])
    rem = myrows - nz * 2 * BCH
    pltpu.sync_copy(pay.at[pl.ds(0, rem)],
                    acc_sh.at[pl.ds(s * myrows + nz * 2 * BCH, rem)])

    gvs = tuple(gvv[h] for h in range(H))
    cvs = tuple(cvv[h] for h in range(H))

    idxs = (idx0, idx1)
    isems = (isem0, isem1)
    ssems = (ssem0, ssem1)
    dsems = (dsem0, dsem1)
    lsems = (lsem0, lsem1)
    row0 = wid * NCH
    ebase = wid * EWP

    def _issue_gathers(b, ch):
        off = b * BCH
        eoff = ebase + ch * BCH
        pltpu.async_copy(stab_hbm.at[idxs[b].at[0]],
                         srows.at[pl.ds(off, BCH)], ssems[b])
        pltpu.async_copy(dtab_hbm.at[idxs[b].at[1]],
                         drows.at[pl.ds(off, BCH)], dsems[b])
        pltpu.async_copy(ek_hbm.at[pl.ds(eoff, BCH)],
                         ekb.at[pl.ds(off, BCH)], lsems[b])
        pltpu.async_copy(ev_hbm.at[pl.ds(eoff, BCH)],
                         evb.at[pl.ds(off, BCH)], lsems[b])
        pltpu.async_copy(ew_hbm.at[pl.ds(eoff, BCH)],
                         ewb.at[pl.ds(off, BCH)], lsems[b])
        pltpu.async_copy(ss_hbm.at[pl.ds(eoff, BCH)],
                         ssb.at[pl.ds(off, BCH)], lsems[b])

    def _wait_gathers(b):
        off = b * BCH
        pltpu.make_async_copy(stab_hbm.at[pl.ds(0, BCH)],
                              srows.at[pl.ds(off, BCH)], ssems[b]).wait()
        pltpu.make_async_copy(dtab_hbm.at[pl.ds(0, BCH)],
                              drows.at[pl.ds(off, BCH)], dsems[b]).wait()
        pltpu.make_async_copy(ek_hbm.at[pl.ds(0, BCH)],
                              ekb.at[pl.ds(off, BCH)], lsems[b]).wait()
        pltpu.make_async_copy(ev_hbm.at[pl.ds(0, BCH)],
                              evb.at[pl.ds(off, BCH)], lsems[b]).wait()
        pltpu.make_async_copy(ew_hbm.at[pl.ds(0, BCH)],
                              ewb.at[pl.ds(off, BCH)], lsems[b]).wait()
        pltpu.make_async_copy(ss_hbm.at[pl.ds(0, BCH)],
                              ssb.at[pl.ds(off, BCH)], lsems[b]).wait()

    plsc.subcore_barrier()

    pltpu.sync_copy(idx_hbm.at[row0], idx0)
    pltpu.async_copy(idx_hbm.at[row0 + 1], idx1, isem1)
    _issue_gathers(0, 0)

    def _edge(boff, e):
        r = boff + e
        dq2 = srows[r, pl.ds(272, 16)]
        dvec = drows[r, pl.ds(256, 16)]
        evec = ssb[r, pl.ds(0, 16)]
        mu = jnp.full((16,), (evec[0] + dq2[8] + dvec[0]) * (1.0 / D))
        msq = jnp.full((16,), (evec[1] + dq2[9] + dvec[1]) * (1.0 / D))
        rstd = _rsqrt16(msq - mu * mu + 1e-5)
        sv = jnp.zeros((16,), jnp.float32)
        for h in range(H):
            qh = srows[r, pl.ds(h * 16, 16)]
            kh = ekb[r, pl.ds(h * 16, 16)] + drows[r, pl.ds(h * 16, 16)]
            sh = jnp.sum(qh * kh)
            sv = jnp.where(lanes == h, jnp.full((16,), sh), sv)
        dq1 = srows[r, pl.ds(256, 16)]
        score = (sv + dq1 - mu * dq2) * rstd
        ex = jnp.where(lanes < H, jnp.exp(score), 0.0)
        pay[r, pl.ds(128, 16)] = ex
        rm = rstd * mu
        for h in range(H):
            Uh = (evb[r, pl.ds(h * 16, 16)]
                  + drows[r, pl.ds(128 + h * 16, 16)]
                  + srows[r, pl.ds(128 + h * 16, 16)])
            vvh = rstd * Uh - rm * gvs[h] + cvs[h]
            exh = jnp.full((16,), ex[h])
            pay[r, pl.ds(h * 16, 16)] = exh * ewb[r, pl.ds(h * 16, 16)] * vvh

    def _pair(it, _):
        for b in (0, 1):
            ch = 2 * it + b
            nb = 1 - b

            @pl.when(ch + 1 < NCH)
            def _prep():
                pltpu.make_async_copy(idx_hbm.at[row0], idxs[nb],
                                      isems[nb]).wait()
                _issue_gathers(nb, ch + 1)

            _wait_gathers(b)
            boff = b * BCH

            def _edge_b(e, _, boff=boff):
                _edge(boff, e)
                return 0
            lax.fori_loop(0, BCH, _edge_b, 0)
            pltpu.sync_copy(pay.at[pl.ds(boff, BCH)],
                            acc_sh.at[idxs[b].at[0]], add=True)

            @pl.when(ch + 2 < NCH)
            def _pref_idx():
                pltpu.async_copy(idx_hbm.at[row0 + ch + 2], idxs[b], isems[b])
        return 0

    lax.fori_loop(0, NCH // 2, _pair, 0)
    plsc.subcore_barrier()
    pltpu.sync_copy(acc_sh.at[pl.ds(s * myrows, myrows)],
                    acc_hbm.at[c, pl.ds(s * myrows, myrows)])


def _stage_b(idx3, stab, dtab, ek, ev, ew, ss, gv, cv):
    mesh = plsc.VectorSubcoreMesh(core_axis_name="c", subcore_axis_name="s")
    kfn = pl.kernel(
        _b_body,
        mesh=mesh,
        compiler_params=pltpu.CompilerParams(
            needs_layout_passes=False, use_tc_tiling_on_sc=False),
        out_type=jax.ShapeDtypeStruct((2, NPAD, PROW), jnp.float32),
        scratch_types=[
            pltpu.VMEM((2, BCH), jnp.int32),
            pltpu.VMEM((2, BCH), jnp.int32),
            pltpu.VMEM((2 * BCH, SROW), jnp.float32),
            pltpu.VMEM((2 * BCH, DROW), jnp.float32),
            pltpu.VMEM((2 * BCH, H * C), jnp.float32),
            pltpu.VMEM((2 * BCH, H * C), jnp.float32),
            pltpu.VMEM((2 * BCH, H * C), jnp.float32),
            pltpu.VMEM((2 * BCH, CE), jnp.float32),
            pltpu.VMEM((2 * BCH, PROW), jnp.float32),
            pltpu.VMEM((H, 16), jnp.float32),
            pltpu.VMEM((H, 16), jnp.float32),
            pltpu.VMEM_SHARED((NPAD, PROW), jnp.float32),
            pltpu.SemaphoreType.DMA,
            pltpu.SemaphoreType.DMA,
            pltpu.SemaphoreType.DMA,
            pltpu.SemaphoreType.DMA,
            pltpu.SemaphoreType.DMA,
            pltpu.SemaphoreType.DMA,
            pltpu.SemaphoreType.DMA,
            pltpu.SemaphoreType.DMA,
        ],
    )
    return kfn(idx3, stab, dtab, ek, ev, ew, ss, gv, cv)


def _c_body(acc_ref, r_ref, wo1_ref, bo1_ref, wo2_ref, bo2_ref, out_ref):
    asum = acc_ref[0] + acc_ref[1]
    num = asum[:, 0:128]
    den = asum[:, 128:136]
    denb = jnp.dot(den, r_ref[...], preferred_element_type=jnp.float32)
    out0 = jnp.where(denb != 0.0, num / denb, 0.0)
    h1 = jnp.dot(out0, wo1_ref[...], preferred_element_type=jnp.float32) + bo1_ref[...]
    h1 = h1 * (1.0 / (1.0 + jnp.exp(-h1)))
    out_ref[...] = jnp.dot(h1, wo2_ref[...], preferred_element_type=jnp.float32) + bo2_ref[...]


def _stage_c(acc, Rm, wo1, bo1, wo2, bo2, interpret=False):
    bo1 = bo1.reshape(1, 2 * CZ)
    bo2 = bo2.reshape(1, CZ)
    full = lambda shp: pl.BlockSpec(shp, lambda i: (0,) * len(shp))
    return pl.pallas_call(
        _c_body,
        grid=(NBLK,),
        in_specs=[
            pl.BlockSpec((2, NB, PROW), lambda i: (0, i, 0)),
            full((H, H * C)),
            full((CZ, 2 * CZ)), full((1, 2 * CZ)),
            full((2 * CZ, CZ)), full((1, CZ)),
        ],
        out_specs=pl.BlockSpec((NB, CZ), lambda i: (i, 0)),
        out_shape=jax.ShapeDtypeStruct((N, CZ), jnp.float32),
        interpret=interpret,
    )(acc, Rm, wo1, bo1, wo2, bo2)


def kernel(x, p, edge_index, edge_attr, ln_gamma, ln_beta, Wq, bq, Wk, bk,
           Wv, bv, We, Wo1, bo1, Wo2, bo2):
    f32 = jnp.float32
    Wkg = ln_gamma[:, None] * Wk
    Wvg = ln_gamma[:, None] * Wv
    scale = f32(1.0 / np.sqrt(C))
    wq4 = Wq * scale
    bq4 = (bq * scale).reshape(1, H * C)
    wka, wkx, wkp = Wkg[:CE], Wkg[CE:CE + CZ], Wkg[CE + CZ:]
    wva, wvx, wvp = Wvg[:CE], Wvg[CE:CE + CZ], Wvg[CE + CZ:]
    gk = (ln_gamma @ Wk).reshape(1, H * C)
    gv = (ln_gamma @ Wv).reshape(H, C)
    cv = (ln_beta @ Wv + bv).reshape(H, C)
    ii = np.arange(H * C)
    S = jnp.asarray((ii[:, None] // C == np.arange(H)[None, :]).astype(np.float32))
    Rm = jnp.asarray((ii[None, :] // C == np.arange(H)[:, None]).astype(np.float32))

    src = jnp.concatenate([edge_index[0].astype(jnp.int32),
                           jnp.full((EPAD - E,), NPAD - 1, jnp.int32)])
    dst = jnp.concatenate([edge_index[1].astype(jnp.int32),
                           jnp.zeros((EPAD - E,), jnp.int32)])
    idx3 = jnp.stack([src.reshape(-1, BCH), dst.reshape(-1, BCH)], axis=1)

    stab, dtab = _stage_a1(x, p, wq4, bq4, wkx, wvx, wkp, wvp, gk, S)
    ek, ev, ew, ss = _stage_a2(edge_attr, wka, wva, We)
    acc = _stage_b(idx3, stab, dtab, ek, ev, ew, ss, gv, cv)
    return _stage_c(acc, Rm, Wo1, bo1, Wo2, bo2)
